# Initial kernel scaffold; baseline (speedup 1.0000x reference)
import jax
import jax.numpy as jnp
from jax.experimental import pallas as pl


def kernel(I_features, T_features):
    raise NotImplementedError("write your pallas kernel here")


# speedup vs baseline: 1.7572x; 1.7572x over previous
import functools

import jax
import jax.numpy as jnp
from jax.experimental import pallas as pl
from jax.experimental.pallas import tpu as pltpu

_SIGMA = 1.0
_B = 1.0
_EPS = 1e-5
_BI = 512


def _cx_kernel(t_ref, i_ref, o_ref, mt_ref, tn_ref, kmax_ref, *, nb, p):
    n = pl.program_id(0)
    ib = pl.program_id(1)

    @pl.when(ib == 0)
    def _prologue():
        t_all = t_ref[...]
        tot = jnp.sum(jnp.sum(t_all, axis=0), axis=1, keepdims=True)
        mt = tot / (t_all.shape[0] * p)
        mt_ref[...] = mt
        tc = t_ref[n] - mt
        tnorm = jnp.sqrt(jnp.sum(tc * tc, axis=0, keepdims=True))
        tn_ref[...] = tc / tnorm
        kmax_ref[...] = jnp.zeros_like(kmax_ref)

    ic = i_ref[0] - mt_ref[...]
    inorm = jnp.sqrt(jnp.sum(ic * ic, axis=0, keepdims=True))
    iu = ic / inorm
    cos = jax.lax.dot_general(
        iu, tn_ref[...],
        dimension_numbers=(((0,), (0,)), ((), ())),
        preferred_element_type=jnp.float32,
    )
    raw = 0.5 - 0.5 * cos
    m = jnp.min(raw, axis=1, keepdims=True) + _EPS
    w = jnp.exp((_B - raw / m) / _SIGMA)
    s = jnp.sum(w, axis=1, keepdims=True)
    cs = w / s
    kmax_ref[...] = jnp.maximum(kmax_ref[...], jnp.max(cs, axis=0, keepdims=True))

    @pl.when(ib == nb - 1)
    def _epilogue():
        cs_mean = jnp.sum(kmax_ref[...]) / p
        o_ref[...] = jnp.full(o_ref.shape, -jnp.log(cs_mean), jnp.float32)


def kernel(I_features, T_features):
    n, c, h, w = I_features.shape
    p = h * w
    i3 = I_features.reshape(n, c, p)
    t3 = T_features.reshape(n, c, p)
    nb = p // _BI

    out = pl.pallas_call(
        functools.partial(_cx_kernel, nb=nb, p=p),
        grid=(n, nb),
        in_specs=[
            pl.BlockSpec((n, c, p), lambda ni, bi: (0, 0, 0)),
            pl.BlockSpec((1, c, _BI), lambda ni, bi: (ni, 0, bi)),
        ],
        out_specs=pl.BlockSpec((1, 1, 128), lambda ni, bi: (ni, 0, 0)),
        out_shape=jax.ShapeDtypeStruct((n, 1, 128), jnp.float32),
        scratch_shapes=[
            pltpu.VMEM((c, 1), jnp.float32),
            pltpu.VMEM((c, p), jnp.float32),
            pltpu.VMEM((1, p), jnp.float32),
        ],
        compiler_params=pltpu.CompilerParams(
            dimension_semantics=("parallel", "arbitrary"),
            vmem_limit_bytes=56 * 1024 * 1024,
        ),
        name="contextual_loss",
    )(t3, i3)
    return jnp.mean(out[:, 0, 0])


# speedup vs baseline: 2.0026x; 1.1396x over previous
import functools

import jax
import jax.numpy as jnp
from jax.experimental import pallas as pl
from jax.experimental.pallas import tpu as pltpu

_SIGMA = 1.0
_B = 1.0
_EPS = 1e-5
_BI = 512


def _cx_kernel(t_ref, i_ref, o_ref, mt_ref, tn_ref, kmax_ref, *, nb, p):
    n = pl.program_id(0)
    ib = pl.program_id(1)

    @pl.when(ib == 0)
    def _prologue():
        t_all = t_ref[...]
        tot = jnp.sum(jnp.sum(t_all, axis=0), axis=1, keepdims=True)
        mt = tot / (t_all.shape[0] * p)
        mt_ref[...] = mt
        tc = t_ref[n] - mt
        tnorm = jnp.sqrt(jnp.sum(tc * tc, axis=0, keepdims=True))
        tn_ref[...] = tc / tnorm
        kmax_ref[...] = jnp.zeros_like(kmax_ref)

    ic = i_ref[0] - mt_ref[...]
    inorm = jnp.sqrt(jnp.sum(ic * ic, axis=0, keepdims=True))
    iu = ic / inorm
    cos = jax.lax.dot_general(
        iu, tn_ref[...],
        dimension_numbers=(((0,), (0,)), ((), ())),
        preferred_element_type=jnp.float32,
    )
    maxcos = jnp.max(cos, axis=1, keepdims=True)
    c2 = 1.0 / (1.0 - maxcos + 2.0 * _EPS)
    c1 = _B - c2
    w = jnp.exp(c1 + c2 * cos)
    s = jnp.sum(w, axis=1, keepdims=True)
    kmax_ref[...] = jnp.maximum(
        kmax_ref[...], jnp.max(w * (1.0 / s), axis=0, keepdims=True))

    @pl.when(ib == nb - 1)
    def _epilogue():
        cs_mean = jnp.sum(kmax_ref[...]) / p
        o_ref[...] = jnp.full(o_ref.shape, -jnp.log(cs_mean), jnp.float32)


def kernel(I_features, T_features):
    n, c, h, w = I_features.shape
    p = h * w
    i3 = I_features.reshape(n, c, p)
    t3 = T_features.reshape(n, c, p)
    nb = p // _BI

    out = pl.pallas_call(
        functools.partial(_cx_kernel, nb=nb, p=p),
        grid=(n, nb),
        in_specs=[
            pl.BlockSpec((n, c, p), lambda ni, bi: (0, 0, 0)),
            pl.BlockSpec((1, c, _BI), lambda ni, bi: (ni, 0, bi)),
        ],
        out_specs=pl.BlockSpec((1, 1, 128), lambda ni, bi: (ni, 0, 0)),
        out_shape=jax.ShapeDtypeStruct((n, 1, 128), jnp.float32),
        scratch_shapes=[
            pltpu.VMEM((c, 1), jnp.float32),
            pltpu.VMEM((c, p), jnp.float32),
            pltpu.VMEM((1, p), jnp.float32),
        ],
        compiler_params=pltpu.CompilerParams(
            dimension_semantics=("parallel", "arbitrary"),
            vmem_limit_bytes=56 * 1024 * 1024,
        ),
        name="contextual_loss",
    )(t3, i3)
    return jnp.mean(out[:, 0, 0])
